# baseline (speedup 1.0000x reference)
import functools

import jax
import jax.numpy as jnp
from jax import lax
from jax.experimental import pallas as pl
from jax.experimental.pallas import tpu as pltpu
from jax.experimental.pallas import tpu_sc as plsc

N = 10000
NP = 10240
E = 320000
D = 128
B = 16
L = 64

NC = 2
NS = 16
CHUNK = 128
EP = 327680
ROWS = EP // CHUNK
RPT = ROWS // (NC * NS)
NPT = NP // NS

_f32 = jnp.float32
_i32 = jnp.int32


def _sc_mesh():
    return plsc.VectorSubcoreMesh(core_axis_name="c", subcore_axis_name="s")


def _sc_degrees(allp, ones, znodes):
    RPT2 = ROWS // NS

    @functools.partial(
        pl.kernel,
        out_type=jax.ShapeDtypeStruct((NC, NP, D), _f32),
        mesh=_sc_mesh(),
        scratch_types=[
            pltpu.VMEM((RPT2, CHUNK), _i32),
            pltpu.VMEM((CHUNK, D), _f32),
            pltpu.VMEM_SHARED((NP, D), _f32),
            pltpu.SemaphoreType.DMA,
        ],
    )
    def k(allp_h, ones_h, znodes_h, deg_h, iv, ones_v, deg_s, sem):
        G = 16
        c = lax.axis_index("c")
        s = lax.axis_index("s")
        r0 = s * NPT
        pltpu.sync_copy(znodes_h.at[pl.ds(r0, NPT)], deg_s.at[pl.ds(r0, NPT)])
        pltpu.sync_copy(ones_h, ones_v)
        base = c * ROWS + s * RPT2
        pltpu.sync_copy(allp_h.at[pl.ds(base, RPT2)], iv)
        plsc.subcore_barrier()

        def body(i, carry):
            descs = [pltpu.async_copy(ones_v, deg_s.at[iv.at[i * G + g]],
                                      sem, add=True) for g in range(G)]
            for d in descs:
                d.wait()
            return carry

        lax.fori_loop(0, RPT2 // G, body, 0)
        plsc.subcore_barrier()
        pltpu.sync_copy(deg_s.at[pl.ds(r0, NPT)], deg_h.at[c, pl.ds(r0, NPT)])

    return k(allp, ones, znodes)


def _sc_message(p, allp, znodes):
    @functools.partial(
        pl.kernel,
        out_type=jax.ShapeDtypeStruct((NC, NP, D), _f32),
        mesh=_sc_mesh(),
        scratch_types=[
            pltpu.VMEM((RPT // 2, CHUNK), _i32),
            pltpu.VMEM((RPT // 2, CHUNK), _i32),
            pltpu.VMEM((CHUNK, D), _f32),
            pltpu.VMEM((CHUNK, D), _f32),
            pltpu.VMEM_SHARED((NP, D), _f32),
            pltpu.SemaphoreType.DMA,
            pltpu.SemaphoreType.DMA,
            pltpu.SemaphoreType.DMA,
            pltpu.SemaphoreType.DMA,
        ],
    )
    def k(p_h, allp_h, znodes_h, agg_h,
          isv, idv, msg0, msg1, acc_s, g0, g1, s0, s1):
        HB = RPT // 2
        c = lax.axis_index("c")
        s = lax.axis_index("s")
        r0 = s * NPT
        pltpu.sync_copy(znodes_h.at[pl.ds(r0, NPT)], acc_s.at[pl.ds(r0, NPT)])
        base = c * (ROWS // NC) + s * RPT
        pltpu.sync_copy(allp_h.at[pl.ds(base, HB)], isv)
        pltpu.sync_copy(allp_h.at[pl.ds(ROWS + base, HB)], idv)
        plsc.subcore_barrier()

        def body(i, carry):
            j = i * 2
            jm = lax.rem(j, HB)

            @pl.when(i > 0)
            def _():
                pltpu.make_async_copy(msg0, acc_s.at[idv.at[jm]], s0).wait()
                pltpu.make_async_copy(msg1, acc_s.at[idv.at[jm]], s1).wait()

            @pl.when(j == HB)
            def _():
                pltpu.sync_copy(allp_h.at[pl.ds(base + HB, HB)], isv)
                pltpu.sync_copy(allp_h.at[pl.ds(ROWS + base + HB, HB)], idv)

            d0 = pltpu.async_copy(p_h.at[isv.at[jm]], msg0, g0)
            d1 = pltpu.async_copy(p_h.at[isv.at[jm + 1]], msg1, g1)
            d0.wait()
            pltpu.async_copy(msg0, acc_s.at[idv.at[jm]], s0, add=True)
            d1.wait()
            pltpu.async_copy(msg1, acc_s.at[idv.at[jm + 1]], s1, add=True)
            return carry

        lax.fori_loop(0, RPT // 2, body, 0)
        pltpu.make_async_copy(msg0, acc_s.at[idv.at[HB - 2]], s0).wait()
        pltpu.make_async_copy(msg1, acc_s.at[idv.at[HB - 1]], s1).wait()
        plsc.subcore_barrier()
        pltpu.sync_copy(acc_s.at[pl.ds(r0, NPT)], agg_h.at[c, pl.ds(r0, NPT)])

    return k(p, allp, znodes)


def _tc_p1(featp, W1, degs):
    R = 2560

    def body(feat_ref, w_ref, deg_ref, out_ref):
        d = deg_ref[0][:, 0:1]
        csrc = lax.rsqrt(jnp.maximum(d, 1.0))
        out_ref[...] = jnp.dot(feat_ref[...], w_ref[...],
                               preferred_element_type=_f32) * csrc

    return pl.pallas_call(
        body,
        grid=(NP // R,),
        in_specs=[pl.BlockSpec((R, D), lambda i: (i, 0)),
                  pl.BlockSpec((D, D), lambda i: (0, 0)),
                  pl.BlockSpec((1, R, D), lambda i: (0, i, 0))],
        out_specs=pl.BlockSpec((R, D), lambda i: (i, 0)),
        out_shape=jax.ShapeDtypeStruct((NP, D), _f32),
    )(featp, W1, degs)


def _tc_mid(agg, degs, W2, b1):
    R = 2560

    def body(agg_ref, deg_ref, w_ref, b_ref, out_ref):
        a = agg_ref[0] + agg_ref[1]
        do = deg_ref[0][:, 0:1]
        di = deg_ref[1][:, 0:1]
        cdst = lax.rsqrt(jnp.maximum(di, 1.0))
        csrc = lax.rsqrt(jnp.maximum(do, 1.0))
        h = jnp.maximum(a * cdst + b_ref[...], 0.0)
        out_ref[...] = jnp.dot(h, w_ref[...],
                               preferred_element_type=_f32) * csrc

    return pl.pallas_call(
        body,
        grid=(NP // R,),
        in_specs=[pl.BlockSpec((NC, R, D), lambda i: (0, i, 0)),
                  pl.BlockSpec((NC, R, D), lambda i: (0, i, 0)),
                  pl.BlockSpec((D, D), lambda i: (0, 0)),
                  pl.BlockSpec((1, D), lambda i: (0, 0))],
        out_specs=pl.BlockSpec((R, D), lambda i: (i, 0)),
        out_shape=jax.ShapeDtypeStruct((NP, D), _f32),
    )(agg, degs, W2, b1)


def _tc_readout(agg, degs, b2, gid3, W_out, b_out):
    R = 2048
    steps = NP // R

    def body(agg_ref, degi_ref, b_ref, gid_ref, wo_ref, bo_ref, out_ref,
             sums, cnts):
        i = pl.program_id(0)

        @pl.when(i == 0)
        def _():
            sums[...] = jnp.zeros((B, D), _f32)
            cnts[...] = jnp.zeros((B, D), _f32)

        a = agg_ref[0] + agg_ref[1]
        di = degi_ref[0][:, 0:1]
        cdst = lax.rsqrt(jnp.maximum(di, 1.0))
        h = a * cdst + b_ref[...]
        ids = gid_ref[0, 0]
        onehot = (ids[:, None] ==
                  lax.broadcasted_iota(_i32, (R, B), 1)).astype(_f32)
        sums[...] += lax.dot_general(onehot, h, (((0,), (0,)), ((), ())),
                                     preferred_element_type=_f32)
        cnts[...] += jnp.broadcast_to(jnp.sum(onehot, axis=0)[:, None], (B, D))

        @pl.when(i == steps - 1)
        def _():
            hg = sums[...] / jnp.maximum(cnts[...], 1.0)
            out_ref[...] = jnp.dot(hg, wo_ref[...],
                                   preferred_element_type=_f32) + bo_ref[...]

    return pl.pallas_call(
        body,
        grid=(steps,),
        in_specs=[pl.BlockSpec((NC, R, D), lambda i: (0, i, 0)),
                  pl.BlockSpec((1, R, D), lambda i: (1, i, 0)),
                  pl.BlockSpec((1, D), lambda i: (0, 0)),
                  pl.BlockSpec((1, 1, R), lambda i: (i, 0, 0)),
                  pl.BlockSpec((D, L), lambda i: (0, 0)),
                  pl.BlockSpec((1, L), lambda i: (0, 0))],
        out_specs=pl.BlockSpec((B, L), lambda i: (0, 0)),
        out_shape=jax.ShapeDtypeStruct((B, L), _f32),
        scratch_shapes=[pltpu.VMEM((B, D), _f32), pltpu.VMEM((B, D), _f32)],
        compiler_params=pltpu.CompilerParams(
            dimension_semantics=("arbitrary",)),
    )(agg, degs, b2, gid3, W_out, b_out)


def kernel(feat, edge_index, graph_ids, W1, b1, W2, b2, W_out, b_out):
    src = edge_index[0].astype(_i32)
    dst = edge_index[1].astype(_i32)
    pad = N + jnp.arange(EP - E, dtype=_i32) % (NP - N)
    allp = jnp.concatenate([src, pad, dst, pad]).reshape(2 * ROWS, CHUNK)
    featp = jnp.pad(feat, ((0, NP - N), (0, 0)))
    znodes = jnp.zeros((NP, D), _f32)
    ones = jnp.ones((CHUNK, D), _f32)
    gidp = jnp.pad(graph_ids.astype(_i32), (0, NP - N), constant_values=B)
    gid3 = gidp.reshape(NP // 2048, 1, 2048)

    degs = _sc_degrees(allp, ones, znodes)
    p1 = _tc_p1(featp, W1, degs)
    agg1 = _sc_message(p1, allp, znodes)
    p2 = _tc_mid(agg1, degs, W2, b1.reshape(1, D))
    agg2 = _sc_message(p2, allp, znodes)
    return _tc_readout(agg2, degs, b2.reshape(1, D), gid3,
                       W_out, b_out.reshape(1, L))
